# Initial kernel scaffold; baseline (speedup 1.0000x reference)
import jax
import jax.numpy as jnp
from jax.experimental import pallas as pl


def kernel(edge_index, x):
    raise NotImplementedError("write your pallas kernel here")


# speedup vs baseline: 5.3955x; 5.3955x over previous
import functools

import jax
import jax.numpy as jnp
from jax import lax
from jax.experimental import pallas as pl
from jax.experimental.pallas import tpu as pltpu
from jax.experimental.pallas import tpu_sc as plsc

_N = 10000
_E = 320000
_D = 128

_NC = 2
_NS = 16
_NW = _NC * _NS
_EPW = _E // _NW
_C = 80
_NCHUNK = _EPW // _C
_NP = 10240
_RPT = _NP // _NS
_ZROWS = 128


def _sc_scatter(src_idx, dst_idx, x, zeros):
    mesh = plsc.VectorSubcoreMesh(core_axis_name="c", subcore_axis_name="s")

    @functools.partial(
        pl.kernel,
        mesh=mesh,
        out_type=jax.ShapeDtypeStruct((_NC, _NP, _D), jnp.float32),
        scratch_types=[
            pltpu.VMEM((_C,), jnp.int32),
            pltpu.VMEM((_C,), jnp.int32),
            pltpu.VMEM((_C, _D), jnp.float32),
            pltpu.VMEM((_ZROWS, _D), jnp.float32),
            pltpu.VMEM_SHARED((_NP, _D), jnp.float32),
            pltpu.SemaphoreType.DMA,
        ],
    )
    def k(src_hbm, dst_hbm, x_hbm, z_hbm, out_hbm,
          sidx, didx, rows, zbuf, acc, sem):
        cid = lax.axis_index("c")
        sid = lax.axis_index("s")
        wid = sid * _NC + cid

        base_row = sid * _RPT
        pltpu.sync_copy(z_hbm, zbuf)

        def zcopy(j, carry):
            pltpu.sync_copy(zbuf, acc.at[pl.ds(base_row + j * _ZROWS, _ZROWS)])
            return carry
        lax.fori_loop(0, _RPT // _ZROWS, zcopy, 0)
        plsc.subcore_barrier()

        ebase = wid * _EPW

        def chunk(j, carry):
            off = pl.multiple_of(ebase + j * _C, 8)
            pltpu.sync_copy(src_hbm.at[pl.ds(off, _C)], sidx)
            pltpu.sync_copy(dst_hbm.at[pl.ds(off, _C)], didx)
            pltpu.async_copy(x_hbm.at[sidx], rows, sem).wait()
            pltpu.sync_copy(rows, acc.at[didx], add=True)
            return carry
        lax.fori_loop(0, _NCHUNK, chunk, 0)
        plsc.subcore_barrier()

        pltpu.sync_copy(acc.at[pl.ds(base_row, _RPT)],
                        out_hbm.at[cid, pl.ds(base_row, _RPT)])

    return k(src_idx, dst_idx, x, zeros)


def _tc_add(p0, p1):
    blk = 1000

    def body(a_ref, b_ref, o_ref):
        o_ref[...] = a_ref[...] + b_ref[...]

    return pl.pallas_call(
        body,
        grid=(_N // blk,),
        in_specs=[pl.BlockSpec((blk, _D), lambda i: (i, 0)),
                  pl.BlockSpec((blk, _D), lambda i: (i, 0))],
        out_specs=pl.BlockSpec((blk, _D), lambda i: (i, 0)),
        out_shape=jax.ShapeDtypeStruct((_N, _D), jnp.float32),
    )(p0, p1)


def kernel(edge_index, x):
    ei = edge_index.astype(jnp.int32)
    src = ei[0]
    dst = ei[1]
    zeros = jnp.zeros((_ZROWS, _D), jnp.float32)
    partials = _sc_scatter(src, dst, x, zeros)
    return _tc_add(partials[0], partials[1])


# speedup vs baseline: 10.7716x; 1.9964x over previous
import functools

import jax
import jax.numpy as jnp
from jax import lax
from jax.experimental import pallas as pl
from jax.experimental.pallas import tpu as pltpu
from jax.experimental.pallas import tpu_sc as plsc

_N = 10000
_E = 320000
_D = 128

_NC = 2
_NS = 16
_NW = _NC * _NS
_EPW = _E // _NW
_C = 80
_NCHUNK = _EPW // _C
_NP = 10240
_RPT = _NP // _NS
_ZROWS = 128


def _sc_scatter(src3, dst3, x, zeros):
    mesh = plsc.VectorSubcoreMesh(core_axis_name="c", subcore_axis_name="s")

    @functools.partial(
        pl.kernel,
        mesh=mesh,
        out_type=jax.ShapeDtypeStruct((_NC, _NP, _D), jnp.float32),
        scratch_types=[
            pltpu.VMEM((_NCHUNK, _C), jnp.int32),
            pltpu.VMEM((_C,), jnp.int32),
            pltpu.VMEM((_C,), jnp.int32),
            pltpu.VMEM((_C, _D), jnp.float32),
            pltpu.VMEM((_C, _D), jnp.float32),
            pltpu.VMEM_SHARED((_NP, _D), jnp.float32),
            pltpu.SemaphoreType.DMA,
            pltpu.SemaphoreType.DMA,
            pltpu.SemaphoreType.DMA,
            pltpu.SemaphoreType.DMA,
            pltpu.SemaphoreType.DMA,
            pltpu.SemaphoreType.DMA,
        ],
    )
    def k(src_hbm, dst_hbm, x_hbm, z_hbm, out_hbm,
          sidx, didx0, didx1, rows0, rows1, acc, g0, g1, s0, s1, i0, i1):
        cid = lax.axis_index("c")
        sid = lax.axis_index("s")
        wid = sid * _NC + cid
        base_row = sid * _RPT

        def zcopy(j, carry):
            pltpu.sync_copy(z_hbm, acc.at[pl.ds(base_row + j * _ZROWS, _ZROWS)])
            return carry
        lax.fori_loop(0, _RPT // _ZROWS, zcopy, 0)
        pltpu.sync_copy(src_hbm.at[wid], sidx)
        plsc.subcore_barrier()

        def gather(j, buf, sem):
            return pltpu.async_copy(x_hbm.at[sidx.at[j]], buf, sem)

        def scat(dbuf, buf, sem):
            return pltpu.async_copy(buf, acc.at[dbuf], sem, add=True)

        def dfetch(j, dbuf, sem):
            return pltpu.async_copy(dst_hbm.at[wid, j], dbuf, sem)

        dfetch(0, didx0, i0).wait()
        gather(0, rows0, g0).wait()
        sc0 = scat(didx0, rows0, s0)
        df1 = dfetch(1, didx1, i1)

        def pair(i, carry):
            a = 2 * i + 1
            b = 2 * i + 2
            ga = gather(a, rows1, g1)
            sc0.wait()
            dfb = dfetch(b, didx0, i0)
            gb = gather(b, rows0, g0)
            ga.wait()
            df1.wait()
            sa = scat(didx1, rows1, s1)
            gb.wait()
            dfb.wait()
            sa.wait()
            nxt = jnp.minimum(a + 2, _NCHUNK - 1)
            dfetch(nxt, didx1, i1)
            scat(didx0, rows0, s0)
            return carry
        lax.fori_loop(0, (_NCHUNK - 1) // 2, pair, 0)
        df1.wait()
        sc0.wait()
        plsc.subcore_barrier()

        pltpu.sync_copy(acc.at[pl.ds(base_row, _RPT)],
                        out_hbm.at[cid, pl.ds(base_row, _RPT)])

    return k(src3, dst3, x, zeros)


def _tc_add(p0, p1):
    blk = 1000

    def body(a_ref, b_ref, o_ref):
        o_ref[...] = a_ref[...] + b_ref[...]

    return pl.pallas_call(
        body,
        grid=(_N // blk,),
        in_specs=[pl.BlockSpec((blk, _D), lambda i: (i, 0)),
                  pl.BlockSpec((blk, _D), lambda i: (i, 0))],
        out_specs=pl.BlockSpec((blk, _D), lambda i: (i, 0)),
        out_shape=jax.ShapeDtypeStruct((_N, _D), jnp.float32),
    )(p0, p1)


def kernel(edge_index, x):
    ei = edge_index.astype(jnp.int32)
    src3 = ei[0].reshape(_NW, _NCHUNK, _C)
    dst3 = ei[1].reshape(_NW, _NCHUNK, _C)
    zeros = jnp.zeros((_ZROWS, _D), jnp.float32)
    partials = _sc_scatter(src3, dst3, x, zeros)
    return _tc_add(partials[0], partials[1])
